# baseline (speedup 1.0000x reference)
import jax
import jax.numpy as jnp
from jax import lax
from jax.experimental import pallas as pl
from jax.experimental.pallas import tpu as pltpu
from jax.experimental.pallas import tpu_sc as plsc

VOCAB = 64
T = 2
NSTEP = T + 1
BATCH = 16384
HIST = 50
PADH = 160

NC, NS = 2, 16
NW = NC * NS
BPW = BATCH // NW
NSTAGE = 16
BSTG = 32
RSTG = BSTG * PADH // 128
NSUB = 8
RSUB = 5
NK = VOCAB // 16


def _sig(x):
    return 1.0 / (1.0 + jnp.exp(-x))


def _body(gidx_hbm, aidx_hbm, gtab_hbm, atab_hbm, gates_hbm,
          obs_hbm, act_hbm,
          gates_v, graw_v, gsh_v, zm1_v, poff1_v, grows_v, oout_v,
          ash_v, apo_v, arows_v, aout_v, sem_g, sem_a):
    wid = lax.axis_index("s") * NC + lax.axis_index("c")
    b0 = wid * BPW

    pltpu.sync_copy(gates_hbm, gates_v)
    t0 = [gates_v[5, pl.ds(16 * k, 16)] for k in range(NK)]
    for r in range(NSTEP + T):
        for k in range(8):
            sl = pl.ds(16 * k, 16)
            gates_v[r, sl] = _sig(gates_v[r, sl])
    for st in range(NSTEP):
        for k in range(NK):
            sl = pl.ds(16 * k, 16)
            gates_v[5 + st, sl] = gates_v[st, sl] * t0[k]

    lanes = lax.iota(jnp.int32, 16)
    tailm = lanes >= 14
    one = jnp.float32(1.0)
    zero = jnp.float32(0.0)

    def stage(sch, carry):
        roff = pl.multiple_of((wid * NSTAGE + sch) * RSTG, 8)
        pltpu.sync_copy(gidx_hbm.at[pl.ds(roff, RSTG)], graw_v)

        def prep(j, c):
            for k in range(8):
                sl = pl.ds(16 * k, 16)
                iv = graw_v[j, sl]
                gsh_v[j, sl] = lax.shift_right_logical(iv, 1)
                fo = j * 128 + 16 * k
                poff1_v[pl.ds(fo, 16)] = (iv & 1) * 64
                zm1_v[pl.ds(fo, 16)] = jnp.where(iv == 0, one, zero)
            return c

        lax.fori_loop(0, RSTG, prep, jnp.int32(0))

        def sub(sc, c):
            cps = [
                pltpu.async_copy(gtab_hbm.at[gsh_v.at[sc * RSUB + jj]],
                                 grows_v.at[pl.ds(128 * jj, 128)], sem_g)
                for jj in range(RSUB)
            ]
            for cp in cps:
                cp.wait()

            def bloop(bi, c2):
                soff0 = sc * (4 * PADH) + bi * PADH
                orow = (sc % 2) * 4 + bi
                for st in range(NSTEP):
                    soff = soff0 + st * HIST
                    rbase = bi * PADH + st * HIST
                    acc = [jnp.zeros((16,), jnp.float32) for _ in range(NK)]
                    for g in range(3):
                        pv = poff1_v[pl.ds(soff + 16 * g, 16)]
                        for j in range(16):
                            p = pv[j]
                            r = rbase + 16 * g + j
                            for k in range(NK):
                                acc[k] = acc[k] + grows_v[r, pl.ds(p + 16 * k, 16)]
                    pv = poff1_v[pl.ds(soff + 34, 16)]
                    for j in (14, 15):
                        p = pv[j]
                        r = rbase + 34 + j
                        for k in range(NK):
                            acc[k] = acc[k] + grows_v[r, pl.ds(p + 16 * k, 16)]
                    zs = (zm1_v[pl.ds(soff, 16)]
                          + zm1_v[pl.ds(soff + 16, 16)]
                          + zm1_v[pl.ds(soff + 32, 16)]
                          + jnp.where(tailm, zm1_v[pl.ds(soff + 34, 16)], zero))
                    zf = zs[0]
                    for j in range(1, 16):
                        zf = zf + zs[j]
                    for k in range(NK):
                        sl = pl.ds(16 * k, 16)
                        sg = gates_v[st, sl]
                        cv = gates_v[5 + st, sl]
                        term = sg * acc[k] - zf * cv
                        if st == 0:
                            oout_v[orow, sl] = term
                        else:
                            oout_v[orow, sl] = oout_v[orow, sl] + term
                return c2

            lax.fori_loop(0, 4, bloop, jnp.int32(0))

            @pl.when(sc % 2 == 1)
            def _flush():
                off = pl.multiple_of(b0 + sch * BSTG + (sc - 1) * 4, 8)
                pltpu.sync_copy(oout_v, obs_hbm.at[pl.ds(off, 8)])

            return c

        lax.fori_loop(0, NSUB, sub, jnp.int32(0))
        return carry

    lax.fori_loop(0, NSTAGE, stage, jnp.int32(0))

    aroff = pl.multiple_of(wid * 8, 8)
    pltpu.sync_copy(aidx_hbm.at[pl.ds(aroff, 8)], ash_v)
    for j in range(8):
        for k in range(8):
            sl = pl.ds(16 * k, 16)
            iv = ash_v[j, sl]
            apo_v[j, sl] = jnp.where(iv == 0, -1, (iv & 1) * 64)
            ash_v[j, sl] = lax.shift_right_logical(iv, 1)

    def achunk(aci, carry):
        pltpu.async_copy(atab_hbm.at[ash_v.at[aci]], arows_v, sem_a).wait()

        def agroup(gg, c):
            sh = [apo_v[aci, pl.ds(16 * gg, 16)]]
            for j in range(8):
                p0 = sh[0][2 * j]
                p1 = sh[0][2 * j + 1]
                m0 = jnp.where(p0 < 0, zero, one)
                m1 = jnp.where(p1 < 0, zero, one)
                q0 = jnp.maximum(p0, 0)
                q1 = jnp.maximum(p1, 0)
                r0 = 16 * gg + 2 * j
                for k in range(NK):
                    sl = pl.ds(16 * k, 16)
                    h0 = arows_v[r0, pl.ds(q0 + 16 * k, 16)]
                    h1 = arows_v[r0 + 1, pl.ds(q1 + 16 * k, 16)]
                    aout_v[j, sl] = (h0 * gates_v[3, sl] * m0
                                     + h1 * gates_v[4, sl] * m1)
            off = pl.multiple_of(b0, 8) + aci * 64 + gg * 8
            pltpu.sync_copy(aout_v, act_hbm.at[pl.ds(off, 8)])
            return c

        lax.fori_loop(0, 8, agroup, jnp.int32(0))
        return carry

    lax.fori_loop(0, 8, achunk, jnp.int32(0))


@jax.jit
def _run(gidx2, aidx2, gtab2, atab2, gates):
    mesh = plsc.VectorSubcoreMesh(core_axis_name="c", subcore_axis_name="s")
    f = pl.kernel(
        _body,
        out_type=[
            jax.ShapeDtypeStruct((BATCH, 128), jnp.float32),
            jax.ShapeDtypeStruct((BATCH, 128), jnp.float32),
        ],
        mesh=mesh,
        compiler_params=pltpu.CompilerParams(use_tc_tiling_on_sc=True),
        scratch_types=[
            pltpu.VMEM((8, 128), jnp.float32),
            pltpu.VMEM((RSTG, 128), jnp.int32),
            pltpu.VMEM((RSTG, 128), jnp.int32),
            pltpu.VMEM((RSTG * 128,), jnp.float32),
            pltpu.VMEM((RSTG * 128,), jnp.int32),
            pltpu.VMEM((128 * RSUB, 128), jnp.float32),
            pltpu.VMEM((8, 128), jnp.float32),
            pltpu.VMEM((8, 128), jnp.int32),
            pltpu.VMEM((8, 128), jnp.int32),
            pltpu.VMEM((128, 128), jnp.float32),
            pltpu.VMEM((8, 128), jnp.float32),
            pltpu.SemaphoreType.DMA,
            pltpu.SemaphoreType.DMA,
        ],
    )
    return f(gidx2, aidx2, gtab2, atab2, gates)


def kernel(goldstandard, actions, gs_table, obs_gates, act_table, act_write_gate):
    gsr = goldstandard.reshape(BATCH, NSTEP * HIST)
    gidx2 = jnp.pad(gsr, ((0, 0), (0, PADH - NSTEP * HIST))).reshape(-1, 128)
    aidx2 = actions.reshape(-1, 128)
    gtab2 = gs_table.reshape(-1, 128)
    atab2 = act_table.reshape(-1, 128)
    gates = (jnp.zeros((8, 128), jnp.float32)
             .at[0:NSTEP, 0:VOCAB].set(obs_gates)
             .at[NSTEP:NSTEP + T, 0:VOCAB].set(act_write_gate.reshape(T, VOCAB))
             .at[5, 0:VOCAB].set(gs_table[0]))
    obs, act = _run(gidx2, aidx2, gtab2, atab2, gates)
    return (obs[:, :VOCAB], act[:, :VOCAB])
